# baseline (speedup 1.0000x reference)
import functools

import jax
import jax.numpy as jnp
from jax import lax
from jax.experimental import pallas as pl
from jax.experimental.pallas import tpu as pltpu
from jax.experimental.pallas import tpu_sc as plsc

_B = 4096
_L = 20
_H = 64
_V = 1000

_NC = 2
_NS = 16
_NW = _NC * _NS
_BPW = _B // _NW
_G = 16
_NG = _BPW // _G


def _make_sc_pool(nb):
    bpw = nb // _NW

    def body(ids_hbm, table_hbm, out_hbm, table_v, ids_v, out_v):
        wid = lax.axis_index("c") * _NS + lax.axis_index("s")
        base_b = wid * bpw
        pltpu.sync_copy(table_hbm, table_v)
        pltpu.sync_copy(ids_hbm.at[pl.ds(base_b, bpw), :], ids_v)

        def row_body(r, carry):
            v0 = ids_v[r, pl.ds(0, 16)] * _H
            v1 = ids_v[r, pl.ds(16, 16)] * _H
            offs = [v0[i] for i in range(16)] + [v1[i] for i in range(_L - 16)]
            for g in range(_H // 16):
                acc = table_v[pl.ds(offs[0] + g * 16, 16)]
                for l in range(1, _L):
                    acc = acc + table_v[pl.ds(offs[l] + g * 16, 16)]
                out_v[r, pl.ds(g * 16, 16)] = acc
            return carry

        lax.fori_loop(0, bpw, row_body, 0)
        pltpu.sync_copy(out_v, out_hbm.at[pl.ds(base_b, bpw), :])

    return pl.kernel(
        body,
        out_type=jax.ShapeDtypeStruct((nb, _H), jnp.float32),
        mesh=plsc.VectorSubcoreMesh(core_axis_name="c", subcore_axis_name="s"),
        compiler_params=pltpu.CompilerParams(needs_layout_passes=False),
        scratch_types=[
            pltpu.VMEM((_V * _H,), jnp.float32),
            pltpu.VMEM((bpw, 32), jnp.int32),
            pltpu.VMEM((bpw, _H), jnp.float32),
        ],
    )


_NSPLIT = 2
_sc_pool_half = _make_sc_pool(_B // _NSPLIT)


_TB = 128
_NQ = 4


def _make_tc_body(base, aliased):
    def body(*refs):
        if aliased:
            x_ref, w_ref, b_ref, _, out_ref, bc_ref, sems = refs
        else:
            x_ref, w_ref, b_ref, out_ref, bc_ref, sems = refs
        i = pl.program_id(0)
        nsteps = pl.num_programs(0)
        slot = lax.rem(i, _NQ)

        @pl.when(i >= _NQ)
        def _():
            pltpu.make_async_copy(
                bc_ref.at[slot], out_ref.at[pl.ds(0, _TB)], sems.at[slot]
            ).wait()

        x = x_ref[:, :] * (1.0 / _L)
        y = jnp.dot(x, w_ref[:, :], preferred_element_type=jnp.float32)
        y = y + b_ref[:, :]
        for l in range(_L):
            bc_ref[slot, :, l, :] = y
        pltpu.make_async_copy(
            bc_ref.at[slot],
            out_ref.at[pl.ds(base + i * _TB, _TB)],
            sems.at[slot],
        ).start()

        @pl.when(i == nsteps - 1)
        def _():
            for q in range(_NQ):
                pltpu.make_async_copy(
                    bc_ref.at[q], out_ref.at[pl.ds(0, _TB)], sems.at[q]
                ).wait()

    return body


def _tc_project(pooled, W, b2d, base, prev=None):
    nb = pooled.shape[0]
    in_specs = [
        pl.BlockSpec((_TB, _H), lambda i: (i, 0)),
        pl.BlockSpec((_H, _V), lambda i: (0, 0)),
        pl.BlockSpec((1, _V), lambda i: (0, 0)),
    ]
    args = [pooled, W, b2d]
    kwargs = {}
    if prev is not None:
        in_specs.append(pl.BlockSpec(memory_space=pl.ANY))
        args.append(prev)
        kwargs["input_output_aliases"] = {3: 0}
    return pl.pallas_call(
        _make_tc_body(base, prev is not None),
        grid=(nb // _TB,),
        in_specs=in_specs,
        out_specs=pl.BlockSpec(memory_space=pl.ANY),
        out_shape=jax.ShapeDtypeStruct((_B, _L, _V), jnp.float32),
        scratch_shapes=[
            pltpu.VMEM((_NQ, _TB, _L, _V), jnp.float32),
            pltpu.SemaphoreType.DMA((_NQ,)),
        ],
        compiler_params=pltpu.CompilerParams(
            dimension_semantics=("arbitrary",)),
        **kwargs,
    )(*args)


def kernel(input_ids, emb_table, W, b):
    ids_pad = jnp.pad(input_ids.astype(jnp.int32), ((0, 0), (0, 32 - _L)))
    table_flat = emb_table.reshape(-1)
    b2d = b.reshape(1, _V)
    nh = _B // _NSPLIT
    pooled = [
        _sc_pool_half(ids_pad[s * nh:(s + 1) * nh], table_flat)
        for s in range(_NSPLIT)
    ]
    out = _tc_project(pooled[0], W, b2d, 0)
    for s in range(1, _NSPLIT):
        out = _tc_project(pooled[s], W, b2d, s * nh, prev=out)
    return out


# speedup vs baseline: 1.0146x; 1.0146x over previous
import functools

import jax
import jax.numpy as jnp
from jax import lax
from jax.experimental import pallas as pl
from jax.experimental.pallas import tpu as pltpu
from jax.experimental.pallas import tpu_sc as plsc

_B = 4096
_L = 20
_H = 64
_V = 1000

_NC = 2
_NS = 16
_NW = _NC * _NS
_BPW = _B // _NW
_G = 16
_NG = _BPW // _G


def _make_sc_pool(nb):
    bpw = nb // _NW

    def body(ids_hbm, table_hbm, out_hbm, table_v, ids_v, out_v):
        wid = lax.axis_index("c") * _NS + lax.axis_index("s")
        base_b = wid * bpw
        pltpu.sync_copy(table_hbm, table_v)
        pltpu.sync_copy(ids_hbm.at[pl.ds(base_b, bpw), :], ids_v)

        def row_body(r, carry):
            v0 = ids_v[r, pl.ds(0, 16)] * _H
            v1 = ids_v[r, pl.ds(16, 16)] * _H
            offs = [v0[i] for i in range(16)] + [v1[i] for i in range(_L - 16)]
            for g in range(_H // 16):
                acc = table_v[pl.ds(offs[0] + g * 16, 16)]
                for l in range(1, _L):
                    acc = acc + table_v[pl.ds(offs[l] + g * 16, 16)]
                out_v[r, pl.ds(g * 16, 16)] = acc
            return carry

        lax.fori_loop(0, bpw, row_body, 0)
        pltpu.sync_copy(out_v, out_hbm.at[pl.ds(base_b, bpw), :])

    return pl.kernel(
        body,
        out_type=jax.ShapeDtypeStruct((nb, _H), jnp.float32),
        mesh=plsc.VectorSubcoreMesh(core_axis_name="c", subcore_axis_name="s"),
        compiler_params=pltpu.CompilerParams(needs_layout_passes=False),
        scratch_types=[
            pltpu.VMEM((_V * _H,), jnp.float32),
            pltpu.VMEM((bpw, 32), jnp.int32),
            pltpu.VMEM((bpw, _H), jnp.float32),
        ],
    )


_NSPLIT = 1
_sc_pool_half = _make_sc_pool(_B // _NSPLIT)


_TB = 64
_NQ = 4


def _make_tc_body(base, aliased):
    def body(*refs):
        if aliased:
            x_ref, w_ref, b_ref, _, out_ref, bc_ref, sems = refs
        else:
            x_ref, w_ref, b_ref, out_ref, bc_ref, sems = refs
        i = pl.program_id(0)
        nsteps = pl.num_programs(0)
        slot = lax.rem(i, _NQ)

        @pl.when(i >= _NQ)
        def _():
            pltpu.make_async_copy(
                bc_ref.at[slot], out_ref.at[pl.ds(0, _TB)], sems.at[slot]
            ).wait()

        x = x_ref[:, :] * (1.0 / _L)
        y = jnp.dot(x, w_ref[:, :], preferred_element_type=jnp.float32)
        y = y + b_ref[:, :]
        for l in range(_L):
            bc_ref[slot, :, l, :] = y
        pltpu.make_async_copy(
            bc_ref.at[slot],
            out_ref.at[pl.ds(base + i * _TB, _TB)],
            sems.at[slot],
        ).start()

        @pl.when(i == nsteps - 1)
        def _():
            for q in range(_NQ):
                pltpu.make_async_copy(
                    bc_ref.at[q], out_ref.at[pl.ds(0, _TB)], sems.at[q]
                ).wait()

    return body


def _tc_project(pooled, W, b2d, base, prev=None):
    nb = pooled.shape[0]
    in_specs = [
        pl.BlockSpec((_TB, _H), lambda i: (i, 0)),
        pl.BlockSpec((_H, _V), lambda i: (0, 0)),
        pl.BlockSpec((1, _V), lambda i: (0, 0)),
    ]
    args = [pooled, W, b2d]
    kwargs = {}
    if prev is not None:
        in_specs.append(pl.BlockSpec(memory_space=pl.ANY))
        args.append(prev)
        kwargs["input_output_aliases"] = {3: 0}
    return pl.pallas_call(
        _make_tc_body(base, prev is not None),
        grid=(nb // _TB,),
        in_specs=in_specs,
        out_specs=pl.BlockSpec(memory_space=pl.ANY),
        out_shape=jax.ShapeDtypeStruct((_B, _L, _V), jnp.float32),
        scratch_shapes=[
            pltpu.VMEM((_NQ, _TB, _L, _V), jnp.float32),
            pltpu.SemaphoreType.DMA((_NQ,)),
        ],
        compiler_params=pltpu.CompilerParams(
            dimension_semantics=("arbitrary",)),
        **kwargs,
    )(*args)


def kernel(input_ids, emb_table, W, b):
    ids_pad = jnp.pad(input_ids.astype(jnp.int32), ((0, 0), (0, 32 - _L)))
    table_flat = emb_table.reshape(-1)
    b2d = b.reshape(1, _V)
    nh = _B // _NSPLIT
    pooled = [
        _sc_pool_half(ids_pad[s * nh:(s + 1) * nh], table_flat)
        for s in range(_NSPLIT)
    ]
    out = _tc_project(pooled[0], W, b2d, 0)
    for s in range(1, _NSPLIT):
        out = _tc_project(pooled[s], W, b2d, s * nh, prev=out)
    return out
